# baseline (speedup 1.0000x reference)
import jax
import jax.numpy as jnp
from jax import lax
from jax.experimental import pallas as pl
from jax.experimental.pallas import tpu as pltpu
from jax.experimental.pallas import tpu_sc as plsc

N = 10000
E = 160000
NHID = 256
NCLASS = 64
NLAYERS = 8
LAMDA = 1.0
ALPHA = 0.1

NC = 2
NS = 16
L = 16
NT = NC * NS
NPAD = 10240
ROWS_T = NPAD // NT
CHUNK = E // NT
CAP = 5120
SEG = 5120
REG = 34 * SEG
BATCH = 32
NBSEG = SEG // BATCH
CSLOT = 128
AGG_ROWS = ROWS_T + 1
DUMP = ROWS_T
DSLOT = 384

BM = 1000

_mesh = plsc.VectorSubcoreMesh(core_axis_name="c", subcore_axis_name="s")


def _mm_relu_body(x_ref, w_ref, o_ref):
    o_ref[...] = jnp.maximum(
        jnp.dot(x_ref[...], w_ref[...], preferred_element_type=jnp.float32), 0.0)


def _mm_body(x_ref, w_ref, o_ref):
    o_ref[...] = jnp.dot(x_ref[...], w_ref[...], preferred_element_type=jnp.float32)


def _matmul(x, w, relu, out_rows):
    _, k = x.shape
    n = w.shape[1]
    return pl.pallas_call(
        _mm_relu_body if relu else _mm_body,
        grid=(N // BM,),
        in_specs=[
            pl.BlockSpec((BM, k), lambda i: (i, 0)),
            pl.BlockSpec((k, n), lambda i: (0, 0)),
        ],
        out_specs=pl.BlockSpec((BM, n), lambda i: (i, 0)),
        out_shape=jax.ShapeDtypeStruct((out_rows, n), jnp.float32),
    )(x, w)


def _prep_body(adj, srcT, dstT, cntT, dinvT,
               srcin, dstin, src2, dst2, degv, cntv):
    c = lax.axis_index("c")
    s = lax.axis_index("s")
    t = c * NS + s
    iota = lax.iota(jnp.int32, L)
    zeros_i = jnp.zeros((L,), jnp.int32)
    dump_v = jnp.full((L,), DUMP, jnp.int32)
    ones_f = jnp.ones((L,), jnp.float32)
    zeros_f = jnp.zeros((L,), jnp.float32)
    lo = t * ROWS_T

    for kk in range(DSLOT // L):
        degv[pl.ds(kk * L, L)] = zeros_f

    cnt = jnp.int32(0)
    nf = jnp.int32(0)

    def scan_chunk(i, carry):
        ca, _ = carry
        off = i * L
        sv = srcin[pl.ds(off, L)]
        dv = dstin[pl.ds(off, L)]
        valid = (off + iota) < CHUNK
        lc = dv - lo
        mk = valid & (lc >= 0) & (lc < ROWS_T)
        plsc.addupdate_scatter(degv, [lc], ones_f, mask=mk)
        cum = plsc.cumsum(mk.astype(jnp.int32))
        pos = ca + cum - 1
        plsc.store_scatter(src2, [pos], sv, mask=mk)
        plsc.store_scatter(dst2, [pos], lc, mask=mk)
        return (ca + jnp.sum(mk.astype(jnp.int32)), 0)

    nvec = (CHUNK + L - 1) // L
    for q in range(NT):
        pltpu.sync_copy(adj.at[pl.ds(q * CAP, CAP)], srcin)
        pltpu.sync_copy(adj.at[pl.ds(NT * CAP + q * CAP, CAP)], dstin)
        cnt, _ = lax.fori_loop(0, nvec, scan_chunk, (cnt, 0))
        flush = cnt >= SEG

        @pl.when(flush)
        def _():
            off = pl.multiple_of(t * REG + nf * SEG, CSLOT)
            pltpu.sync_copy(src2.at[pl.ds(0, SEG)], srcT.at[pl.ds(off, SEG)])
            pltpu.sync_copy(dst2.at[pl.ds(0, SEG)], dstT.at[pl.ds(off, SEG)])

            def shift(i, _):
                o = i * L
                src2[pl.ds(o, L)] = src2[pl.ds(SEG + o, L)]
                dst2[pl.ds(o, L)] = dst2[pl.ds(SEG + o, L)]
                return 0
            lax.fori_loop(0, SEG // L, shift, 0)

        fi = flush.astype(jnp.int32)
        nf = nf + fi
        cnt = cnt - SEG * fi

    for kk in range(BATCH // L * 2):
        src2[pl.ds(cnt + kk * L, L)] = zeros_i
        dst2[pl.ds(cnt + kk * L, L)] = dump_v
    cp = ((cnt + BATCH - 1) >> 5) << 5
    off = pl.multiple_of(t * REG + nf * SEG, CSLOT)
    pltpu.sync_copy(src2.at[pl.ds(0, SEG)], srcT.at[pl.ds(off, SEG)])
    pltpu.sync_copy(dst2.at[pl.ds(0, SEG)], dstT.at[pl.ds(off, SEG)])
    off2 = pl.multiple_of(t * REG + (nf + 1) * SEG, CSLOT)
    pltpu.sync_copy(src2.at[pl.ds(SEG, SEG)], srcT.at[pl.ds(off2, SEG)])
    pltpu.sync_copy(dst2.at[pl.ds(SEG, SEG)], dstT.at[pl.ds(off2, SEG)])
    total = nf * SEG + cp
    nseg = nf + (cp > 0).astype(jnp.int32) + (cp > SEG).astype(jnp.int32)
    for kk in range(CSLOT // L):
        cntv[pl.ds(kk * L, L)] = jnp.full((L,), total, jnp.int32)
    for kk in range(1, 2):
        cntv[pl.ds(kk * L, L)] = jnp.full((L,), nseg, jnp.int32)
    pltpu.sync_copy(cntv, cntT.at[pl.ds(t * CSLOT, CSLOT)])

    for kk in range(DSLOT // L):
        dv = degv[pl.ds(kk * L, L)]
        degv[pl.ds(kk * L, L)] = 1.0 / (LAMDA * dv + 1.0)
    pltpu.sync_copy(degv, dinvT.at[pl.ds(t * DSLOT, DSLOT)])


_prep = pl.kernel(
    _prep_body,
    compiler_params=pltpu.CompilerParams(needs_layout_passes=False),
    out_type=[
        jax.ShapeDtypeStruct((NT * REG,), jnp.int32),
        jax.ShapeDtypeStruct((NT * REG,), jnp.int32),
        jax.ShapeDtypeStruct((NT * CSLOT,), jnp.int32),
        jax.ShapeDtypeStruct((NT * DSLOT,), jnp.float32),
    ],
    mesh=_mesh,
    scratch_types=[
        pltpu.VMEM((CAP,), jnp.int32),
        pltpu.VMEM((CAP,), jnp.int32),
        pltpu.VMEM((3 * SEG,), jnp.int32),
        pltpu.VMEM((3 * SEG,), jnp.int32),
        pltpu.VMEM((DSLOT,), jnp.float32),
        pltpu.VMEM((CSLOT,), jnp.int32),
    ],
)


def _layer_body(Yin, h0, dinvT, srcT, dstT, cntT, Yout,
                srcv, dstv, cntv, rows_a, rows_b, rows_c, rows_d, dinvv, agg,
                sem, sem2, sem3, sem4):
    c = lax.axis_index("c")
    s = lax.axis_index("s")
    t = c * NS + s
    zeros_f = jnp.zeros((L,), jnp.float32)

    def zrow(j, _):
        for k in range(NHID // L):
            agg[j, pl.ds(k * L, L)] = zeros_f
        return 0
    lax.fori_loop(0, AGG_ROWS, zrow, 0)

    pltpu.sync_copy(cntT.at[pl.ds(t * CSLOT, CSLOT)], cntv)
    total = jnp.max(cntv[pl.ds(0, L)])
    nseg = jnp.max(cntv[pl.ds(L, L)])

    def accumulate(j, buf):
        def acc_group(e2, _):
            dvec = dstv[pl.ds(j * BATCH + e2 * L, L)]
            dvals = [dvec[lane] for lane in range(L)]
            prev = None
            for lane in range(L):
                for kk in range(2):
                    cur = []
                    for k8 in range(8):
                        if prev is not None:
                            pv, pd, pk = prev
                            plsc.addupdate(
                                agg.at[pd, pl.ds(pk * 8 * L + k8 * L, L)],
                                pv[k8])
                        cur.append(
                            buf[e2 * L + lane, pl.ds(kk * 8 * L + k8 * L, L)])
                    prev = (cur, dvals[lane], kk)
            pv, pd, pk = prev
            for k8 in range(8):
                plsc.addupdate(agg.at[pd, pl.ds(pk * 8 * L + k8 * L, L)],
                               pv[k8])
            return 0
        lax.fori_loop(0, BATCH // L, acc_group, 0)

    def seg_loop(sg, _):
        off = pl.multiple_of(t * REG + sg * SEG, CSLOT)
        pltpu.sync_copy(srcT.at[pl.ds(off, SEG)], srcv)
        pltpu.sync_copy(dstT.at[pl.ds(off, SEG)], dstv)
        nb = jnp.minimum(NBSEG, (total - sg * SEG) >> 5)

        bufs = ((rows_a, sem), (rows_b, sem2), (rows_c, sem3), (rows_d, sem4))
        for b in range(3):
            buf, sb = bufs[b]

            @pl.when(b < nb)
            def _():
                pltpu.async_copy(Yin.at[srcv.at[pl.ds(b * BATCH, BATCH)]],
                                 buf, sb)

        def quad_loop(p, _):
            for b in range(4):
                buf, sb = bufs[b]
                nbuf, nsem = bufs[(b + 3) % 4]
                j = p * 4 + b

                @pl.when(j < nb)
                def _():
                    pltpu.make_async_copy(Yin.at[srcv.at[pl.ds(0, BATCH)]],
                                          buf, sb).wait()
                    nxt = jnp.minimum(j + 3, nb - 1) * BATCH

                    @pl.when(j + 3 < nb)
                    def _():
                        pltpu.async_copy(Yin.at[srcv.at[pl.ds(nxt, BATCH)]],
                                         nbuf, nsem)
                    accumulate(j, buf)
            return 0
        lax.fori_loop(0, (NBSEG + 3) // 4, quad_loop, 0)
        return 0
    lax.fori_loop(0, nseg, seg_loop, 0)

    g0 = t * ROWS_T
    pltpu.sync_copy(dinvT.at[pl.ds(t * DSLOT, DSLOT)], dinvv)
    def ublock(u, _):
        g = g0 + u * BATCH
        pltpu.sync_copy(Yin.at[pl.ds(g, BATCH)], rows_a)
        pltpu.sync_copy(h0.at[pl.ds(g, BATCH)], rows_b)

        def upd(q, _):
            dinv = dinvv[pl.ds(u * BATCH + q * L, L)]
            for rr in range(L):
                d = ALPHA * dinv[rr]
                r = q * L + rr

                def upd_half(kk, _):
                    css = [pl.ds(kk * 8 * L + k8 * L, L) for k8 in range(8)]
                    ys = [rows_a[r, cs] for cs in css]
                    hs = [rows_b[r, cs] for cs in css]
                    ags = [agg[u * BATCH + r, cs] for cs in css]
                    for k8 in range(8):
                        nv = ((1.0 - ALPHA) * ys[k8]
                              + d * (LAMDA * ags[k8] + hs[k8]))
                        rows_b[r, css[k8]] = jnp.maximum(nv, 0.0)
                    return 0
                lax.fori_loop(0, 2, upd_half, 0)
            return 0
        lax.fori_loop(0, BATCH // L, upd, 0)
        pltpu.sync_copy(rows_b, Yout.at[pl.ds(g, BATCH)])
        return 0
    lax.fori_loop(0, ROWS_T // BATCH, ublock, 0)


_layer = pl.kernel(
    _layer_body,
    compiler_params=pltpu.CompilerParams(needs_layout_passes=False),
    out_type=jax.ShapeDtypeStruct((NPAD, NHID), jnp.float32),
    mesh=_mesh,
    scratch_types=[
        pltpu.VMEM((SEG,), jnp.int32),
        pltpu.VMEM((SEG,), jnp.int32),
        pltpu.VMEM((CSLOT,), jnp.int32),
        pltpu.VMEM((BATCH, NHID), jnp.float32),
        pltpu.VMEM((BATCH, NHID), jnp.float32),
        pltpu.VMEM((BATCH, NHID), jnp.float32),
        pltpu.VMEM((BATCH, NHID), jnp.float32),
        pltpu.VMEM((DSLOT,), jnp.float32),
        pltpu.VMEM((AGG_ROWS, NHID), jnp.float32),
        pltpu.SemaphoreType.DMA,
        pltpu.SemaphoreType.DMA,
        pltpu.SemaphoreType.DMA,
        pltpu.SemaphoreType.DMA,
    ],
)


def kernel(fea, adj, W_in, W_out):
    h0 = _matmul(fea, W_in, relu=True, out_rows=NPAD)
    adjp = jnp.pad(adj.reshape(2, NT, CHUNK),
                   ((0, 0), (0, 0), (0, CAP - CHUNK))).reshape(-1)
    srcT, dstT, cntT, dinvT = _prep(adjp)
    Y = h0
    for _ in range(NLAYERS):
        Y = _layer(Y, h0, dinvT, srcT, dstT, cntT)
    return _matmul(Y, W_out, relu=False, out_rows=N)


# speedup vs baseline: 1.0291x; 1.0291x over previous
import jax
import jax.numpy as jnp
from jax import lax
from jax.experimental import pallas as pl
from jax.experimental.pallas import tpu as pltpu
from jax.experimental.pallas import tpu_sc as plsc

N = 10000
E = 160000
NHID = 256
NCLASS = 64
NLAYERS = 8
LAMDA = 1.0
ALPHA = 0.1

NC = 2
NS = 16
L = 16
NT = NC * NS
NPAD = 10240
ROWS_T = NPAD // NT
CHUNK = E // NT
CAP = 5120
SEG = 5120
REG = 34 * SEG
BATCH = 32
NBSEG = SEG // BATCH
CSLOT = 128
AGG_ROWS = ROWS_T + 1
DUMP = ROWS_T
DSLOT = 384

BM = 1000

_mesh = plsc.VectorSubcoreMesh(core_axis_name="c", subcore_axis_name="s")


def _mm_relu_body(x_ref, w_ref, o_ref):
    o_ref[...] = jnp.maximum(
        jnp.dot(x_ref[...], w_ref[...], preferred_element_type=jnp.float32), 0.0)


def _mm_body(x_ref, w_ref, o_ref):
    o_ref[...] = jnp.dot(x_ref[...], w_ref[...], preferred_element_type=jnp.float32)


def _matmul(x, w, relu, out_rows):
    _, k = x.shape
    n = w.shape[1]
    return pl.pallas_call(
        _mm_relu_body if relu else _mm_body,
        grid=(N // BM,),
        in_specs=[
            pl.BlockSpec((BM, k), lambda i: (i, 0)),
            pl.BlockSpec((k, n), lambda i: (0, 0)),
        ],
        out_specs=pl.BlockSpec((BM, n), lambda i: (i, 0)),
        out_shape=jax.ShapeDtypeStruct((out_rows, n), jnp.float32),
    )(x, w)


def _prep_body(adj, srcT, dstT, cntT, dinvT,
               srcin, dstin, src2, dst2, degv, cntv):
    c = lax.axis_index("c")
    s = lax.axis_index("s")
    t = c * NS + s
    iota = lax.iota(jnp.int32, L)
    zeros_i = jnp.zeros((L,), jnp.int32)
    dump_v = jnp.full((L,), DUMP, jnp.int32)
    ones_f = jnp.ones((L,), jnp.float32)
    zeros_f = jnp.zeros((L,), jnp.float32)
    lo = t * ROWS_T

    for kk in range(DSLOT // L):
        degv[pl.ds(kk * L, L)] = zeros_f

    cnt = jnp.int32(0)
    nf = jnp.int32(0)

    def scan_chunk(i, carry):
        ca, _ = carry
        off = i * L
        sv = srcin[pl.ds(off, L)]
        dv = dstin[pl.ds(off, L)]
        valid = (off + iota) < CHUNK
        lc = dv - lo
        mk = valid & (lc >= 0) & (lc < ROWS_T)
        plsc.addupdate_scatter(degv, [lc], ones_f, mask=mk)
        cum = plsc.cumsum(mk.astype(jnp.int32))
        pos = ca + cum - 1
        plsc.store_scatter(src2, [pos], sv, mask=mk)
        plsc.store_scatter(dst2, [pos], lc, mask=mk)
        return (ca + jnp.sum(mk.astype(jnp.int32)), 0)

    nvec = (CHUNK + L - 1) // L
    for q in range(NT):
        pltpu.sync_copy(adj.at[pl.ds(q * CAP, CAP)], srcin)
        pltpu.sync_copy(adj.at[pl.ds(NT * CAP + q * CAP, CAP)], dstin)
        cnt, _ = lax.fori_loop(0, nvec, scan_chunk, (cnt, 0))
        flush = cnt >= SEG

        @pl.when(flush)
        def _():
            off = pl.multiple_of(t * REG + nf * SEG, CSLOT)
            pltpu.sync_copy(src2.at[pl.ds(0, SEG)], srcT.at[pl.ds(off, SEG)])
            pltpu.sync_copy(dst2.at[pl.ds(0, SEG)], dstT.at[pl.ds(off, SEG)])

            def shift(i, _):
                o = i * L
                src2[pl.ds(o, L)] = src2[pl.ds(SEG + o, L)]
                dst2[pl.ds(o, L)] = dst2[pl.ds(SEG + o, L)]
                return 0
            lax.fori_loop(0, SEG // L, shift, 0)

        fi = flush.astype(jnp.int32)
        nf = nf + fi
        cnt = cnt - SEG * fi

    for kk in range(BATCH // L * 2):
        src2[pl.ds(cnt + kk * L, L)] = zeros_i
        dst2[pl.ds(cnt + kk * L, L)] = dump_v
    cp = ((cnt + BATCH - 1) >> 5) << 5
    off = pl.multiple_of(t * REG + nf * SEG, CSLOT)
    pltpu.sync_copy(src2.at[pl.ds(0, SEG)], srcT.at[pl.ds(off, SEG)])
    pltpu.sync_copy(dst2.at[pl.ds(0, SEG)], dstT.at[pl.ds(off, SEG)])
    off2 = pl.multiple_of(t * REG + (nf + 1) * SEG, CSLOT)
    pltpu.sync_copy(src2.at[pl.ds(SEG, SEG)], srcT.at[pl.ds(off2, SEG)])
    pltpu.sync_copy(dst2.at[pl.ds(SEG, SEG)], dstT.at[pl.ds(off2, SEG)])
    total = nf * SEG + cp
    nseg = nf + (cp > 0).astype(jnp.int32) + (cp > SEG).astype(jnp.int32)
    for kk in range(CSLOT // L):
        cntv[pl.ds(kk * L, L)] = jnp.full((L,), total, jnp.int32)
    for kk in range(1, 2):
        cntv[pl.ds(kk * L, L)] = jnp.full((L,), nseg, jnp.int32)
    pltpu.sync_copy(cntv, cntT.at[pl.ds(t * CSLOT, CSLOT)])

    for kk in range(DSLOT // L):
        dv = degv[pl.ds(kk * L, L)]
        degv[pl.ds(kk * L, L)] = 1.0 / (LAMDA * dv + 1.0)
    pltpu.sync_copy(degv, dinvT.at[pl.ds(t * DSLOT, DSLOT)])


_prep = pl.kernel(
    _prep_body,
    compiler_params=pltpu.CompilerParams(needs_layout_passes=False),
    out_type=[
        jax.ShapeDtypeStruct((NT * REG,), jnp.int32),
        jax.ShapeDtypeStruct((NT * REG,), jnp.int32),
        jax.ShapeDtypeStruct((NT * CSLOT,), jnp.int32),
        jax.ShapeDtypeStruct((NT * DSLOT,), jnp.float32),
    ],
    mesh=_mesh,
    scratch_types=[
        pltpu.VMEM((CAP,), jnp.int32),
        pltpu.VMEM((CAP,), jnp.int32),
        pltpu.VMEM((3 * SEG,), jnp.int32),
        pltpu.VMEM((3 * SEG,), jnp.int32),
        pltpu.VMEM((DSLOT,), jnp.float32),
        pltpu.VMEM((CSLOT,), jnp.int32),
    ],
)


def _layer_body(Yin, h0, dinvT, srcT, dstT, cntT, Yout,
                srcv, dstv, cntv, rows_a, rows_b, dinvv, agg, sem, sem2):
    c = lax.axis_index("c")
    s = lax.axis_index("s")
    t = c * NS + s
    zeros_f = jnp.zeros((L,), jnp.float32)

    def zrow(j, _):
        for k in range(NHID // L):
            agg[j, pl.ds(k * L, L)] = zeros_f
        return 0
    lax.fori_loop(0, AGG_ROWS, zrow, 0)

    pltpu.sync_copy(cntT.at[pl.ds(t * CSLOT, CSLOT)], cntv)
    total = jnp.max(cntv[pl.ds(0, L)])
    nseg = jnp.max(cntv[pl.ds(L, L)])

    def accumulate(j, buf):
        def acc_group(e2, _):
            dvec = dstv[pl.ds(j * BATCH + e2 * L, L)]
            dvals = [dvec[lane] for lane in range(L)]
            prev = None
            for lane in range(L):
                for kk in range(2):
                    cur = []
                    for k8 in range(8):
                        if prev is not None:
                            pv, pd, pk = prev
                            plsc.addupdate(
                                agg.at[pd, pl.ds(pk * 8 * L + k8 * L, L)],
                                pv[k8])
                        cur.append(
                            buf[e2 * L + lane, pl.ds(kk * 8 * L + k8 * L, L)])
                    prev = (cur, dvals[lane], kk)
            pv, pd, pk = prev
            for k8 in range(8):
                plsc.addupdate(agg.at[pd, pl.ds(pk * 8 * L + k8 * L, L)],
                               pv[k8])
            return 0
        lax.fori_loop(0, BATCH // L, acc_group, 0)

    def seg_loop(sg, _):
        off = pl.multiple_of(t * REG + sg * SEG, CSLOT)
        pltpu.sync_copy(srcT.at[pl.ds(off, SEG)], srcv)
        pltpu.sync_copy(dstT.at[pl.ds(off, SEG)], dstv)
        nb = jnp.minimum(NBSEG, (total - sg * SEG) >> 5)

        @pl.when(nb > 0)
        def _():
            pltpu.async_copy(Yin.at[srcv.at[pl.ds(0, BATCH)]], rows_a, sem)

        def pair_loop(p, _):
            for b, (buf, sb) in enumerate(((rows_a, sem), (rows_b, sem2))):
                j = p * 2 + b

                @pl.when(j < nb)
                def _():
                    pltpu.make_async_copy(Yin.at[srcv.at[pl.ds(0, BATCH)]],
                                          buf, sb).wait()
                    nxt = jnp.minimum(j + 1, nb - 1) * BATCH
                    nbuf = rows_b if b == 0 else rows_a
                    nsem = sem2 if b == 0 else sem

                    @pl.when(j + 1 < nb)
                    def _():
                        pltpu.async_copy(Yin.at[srcv.at[pl.ds(nxt, BATCH)]],
                                         nbuf, nsem)
                    accumulate(j, buf)
            return 0
        lax.fori_loop(0, (NBSEG + 1) // 2, pair_loop, 0)
        return 0
    lax.fori_loop(0, nseg, seg_loop, 0)

    g0 = t * ROWS_T
    pltpu.sync_copy(dinvT.at[pl.ds(t * DSLOT, DSLOT)], dinvv)
    def ublock(u, _):
        g = g0 + u * BATCH
        pltpu.sync_copy(Yin.at[pl.ds(g, BATCH)], rows_a)
        pltpu.sync_copy(h0.at[pl.ds(g, BATCH)], rows_b)

        def upd(q, _):
            dinv = dinvv[pl.ds(u * BATCH + q * L, L)]
            for rr in range(L):
                d = ALPHA * dinv[rr]
                r = q * L + rr

                def upd_half(kk, _):
                    css = [pl.ds(kk * 8 * L + k8 * L, L) for k8 in range(8)]
                    ys = [rows_a[r, cs] for cs in css]
                    hs = [rows_b[r, cs] for cs in css]
                    ags = [agg[u * BATCH + r, cs] for cs in css]
                    for k8 in range(8):
                        nv = ((1.0 - ALPHA) * ys[k8]
                              + d * (LAMDA * ags[k8] + hs[k8]))
                        rows_b[r, css[k8]] = jnp.maximum(nv, 0.0)
                    return 0
                lax.fori_loop(0, 2, upd_half, 0)
            return 0
        lax.fori_loop(0, BATCH // L, upd, 0)
        pltpu.sync_copy(rows_b, Yout.at[pl.ds(g, BATCH)])
        return 0
    lax.fori_loop(0, ROWS_T // BATCH, ublock, 0)


_layer = pl.kernel(
    _layer_body,
    compiler_params=pltpu.CompilerParams(needs_layout_passes=False),
    out_type=jax.ShapeDtypeStruct((NPAD, NHID), jnp.float32),
    mesh=_mesh,
    scratch_types=[
        pltpu.VMEM((SEG,), jnp.int32),
        pltpu.VMEM((SEG,), jnp.int32),
        pltpu.VMEM((CSLOT,), jnp.int32),
        pltpu.VMEM((BATCH, NHID), jnp.float32),
        pltpu.VMEM((BATCH, NHID), jnp.float32),
        pltpu.VMEM((DSLOT,), jnp.float32),
        pltpu.VMEM((AGG_ROWS, NHID), jnp.float32),
        pltpu.SemaphoreType.DMA,
        pltpu.SemaphoreType.DMA,
    ],
)


def kernel(fea, adj, W_in, W_out):
    h0 = _matmul(fea, W_in, relu=True, out_rows=NPAD)
    adjp = jnp.pad(adj.reshape(2, NT, CHUNK),
                   ((0, 0), (0, 0), (0, CAP - CHUNK))).reshape(-1)
    srcT, dstT, cntT, dinvT = _prep(adjp)
    Y = h0
    for _ in range(NLAYERS):
        Y = _layer(Y, h0, dinvT, srcT, dstT, cntT)
    return _matmul(Y, W_out, relu=False, out_rows=N)


# speedup vs baseline: 1.0443x; 1.0148x over previous
import jax
import jax.numpy as jnp
from jax import lax
from jax.experimental import pallas as pl
from jax.experimental.pallas import tpu as pltpu
from jax.experimental.pallas import tpu_sc as plsc

N = 10000
E = 160000
NHID = 256
NCLASS = 64
NLAYERS = 8
LAMDA = 1.0
ALPHA = 0.1

NC = 2
NS = 16
L = 16
NT = NC * NS
NPAD = 10240
ROWS_T = NPAD // NT
CHUNK = E // NT
CAP = 5120
SEG = 5120
REG = 34 * SEG
BATCH = 32
NBSEG = SEG // BATCH
CSLOT = 128
AGG_ROWS = ROWS_T + 1
DUMP = ROWS_T
DSLOT = 384

BM = 1000

_mesh = plsc.VectorSubcoreMesh(core_axis_name="c", subcore_axis_name="s")


def _mm_relu_body(x_ref, w_ref, o_ref):
    o_ref[...] = jnp.maximum(
        jnp.dot(x_ref[...], w_ref[...], preferred_element_type=jnp.float32), 0.0)


def _mm_body(x_ref, w_ref, o_ref):
    o_ref[...] = jnp.dot(x_ref[...], w_ref[...], preferred_element_type=jnp.float32)


def _matmul(x, w, relu, out_rows):
    _, k = x.shape
    n = w.shape[1]
    return pl.pallas_call(
        _mm_relu_body if relu else _mm_body,
        grid=(N // BM,),
        in_specs=[
            pl.BlockSpec((BM, k), lambda i: (i, 0)),
            pl.BlockSpec((k, n), lambda i: (0, 0)),
        ],
        out_specs=pl.BlockSpec((BM, n), lambda i: (i, 0)),
        out_shape=jax.ShapeDtypeStruct((out_rows, n), jnp.float32),
    )(x, w)


def _prep_body(adj, srcT, dstT, cntT, dinvT,
               srcin, dstin, src2, dst2, degv, cntv):
    c = lax.axis_index("c")
    s = lax.axis_index("s")
    t = c * NS + s
    iota = lax.iota(jnp.int32, L)
    zeros_i = jnp.zeros((L,), jnp.int32)
    dump_v = jnp.full((L,), DUMP, jnp.int32)
    ones_f = jnp.ones((L,), jnp.float32)
    zeros_f = jnp.zeros((L,), jnp.float32)
    lo = t * ROWS_T

    for kk in range(DSLOT // L):
        degv[pl.ds(kk * L, L)] = zeros_f

    cnt = jnp.int32(0)
    nf = jnp.int32(0)

    def scan_chunk(i, carry):
        ca, _ = carry
        off = i * L
        sv = srcin[pl.ds(off, L)]
        dv = dstin[pl.ds(off, L)]
        valid = (off + iota) < CHUNK
        lc = dv - lo
        mk = valid & (lc >= 0) & (lc < ROWS_T)
        plsc.addupdate_scatter(degv, [lc], ones_f, mask=mk)
        plsc.store_compressed(src2.at[pl.ds(ca, L)], sv, mask=mk)
        plsc.store_compressed(dst2.at[pl.ds(ca, L)], lc, mask=mk)
        npop = plsc.all_reduce_population_count(mk)
        return (ca + npop[0], 0)

    nvec = (CHUNK + L - 1) // L
    for q in range(NT):
        pltpu.sync_copy(adj.at[pl.ds(q * CAP, CAP)], srcin)
        pltpu.sync_copy(adj.at[pl.ds(NT * CAP + q * CAP, CAP)], dstin)
        cnt, _ = lax.fori_loop(0, nvec, scan_chunk, (cnt, 0))
        flush = cnt >= SEG

        @pl.when(flush)
        def _():
            off = pl.multiple_of(t * REG + nf * SEG, CSLOT)
            pltpu.sync_copy(src2.at[pl.ds(0, SEG)], srcT.at[pl.ds(off, SEG)])
            pltpu.sync_copy(dst2.at[pl.ds(0, SEG)], dstT.at[pl.ds(off, SEG)])

            def shift(i, _):
                o = i * L
                src2[pl.ds(o, L)] = src2[pl.ds(SEG + o, L)]
                dst2[pl.ds(o, L)] = dst2[pl.ds(SEG + o, L)]
                return 0
            lax.fori_loop(0, SEG // L, shift, 0)

        fi = flush.astype(jnp.int32)
        nf = nf + fi
        cnt = cnt - SEG * fi

    for kk in range(BATCH // L * 2):
        src2[pl.ds(cnt + kk * L, L)] = zeros_i
        dst2[pl.ds(cnt + kk * L, L)] = dump_v
    cp = ((cnt + BATCH - 1) >> 5) << 5
    off = pl.multiple_of(t * REG + nf * SEG, CSLOT)
    pltpu.sync_copy(src2.at[pl.ds(0, SEG)], srcT.at[pl.ds(off, SEG)])
    pltpu.sync_copy(dst2.at[pl.ds(0, SEG)], dstT.at[pl.ds(off, SEG)])
    off2 = pl.multiple_of(t * REG + (nf + 1) * SEG, CSLOT)
    pltpu.sync_copy(src2.at[pl.ds(SEG, SEG)], srcT.at[pl.ds(off2, SEG)])
    pltpu.sync_copy(dst2.at[pl.ds(SEG, SEG)], dstT.at[pl.ds(off2, SEG)])
    total = nf * SEG + cp
    nseg = nf + (cp > 0).astype(jnp.int32) + (cp > SEG).astype(jnp.int32)
    for kk in range(CSLOT // L):
        cntv[pl.ds(kk * L, L)] = jnp.full((L,), total, jnp.int32)
    for kk in range(1, 2):
        cntv[pl.ds(kk * L, L)] = jnp.full((L,), nseg, jnp.int32)
    pltpu.sync_copy(cntv, cntT.at[pl.ds(t * CSLOT, CSLOT)])

    for kk in range(DSLOT // L):
        dv = degv[pl.ds(kk * L, L)]
        degv[pl.ds(kk * L, L)] = 1.0 / (LAMDA * dv + 1.0)
    pltpu.sync_copy(degv, dinvT.at[pl.ds(t * DSLOT, DSLOT)])


_prep = pl.kernel(
    _prep_body,
    compiler_params=pltpu.CompilerParams(needs_layout_passes=False),
    out_type=[
        jax.ShapeDtypeStruct((NT * REG,), jnp.int32),
        jax.ShapeDtypeStruct((NT * REG,), jnp.int32),
        jax.ShapeDtypeStruct((NT * CSLOT,), jnp.int32),
        jax.ShapeDtypeStruct((NT * DSLOT,), jnp.float32),
    ],
    mesh=_mesh,
    scratch_types=[
        pltpu.VMEM((CAP,), jnp.int32),
        pltpu.VMEM((CAP,), jnp.int32),
        pltpu.VMEM((3 * SEG,), jnp.int32),
        pltpu.VMEM((3 * SEG,), jnp.int32),
        pltpu.VMEM((DSLOT,), jnp.float32),
        pltpu.VMEM((CSLOT,), jnp.int32),
    ],
)


def _layer_body(Yin, h0, dinvT, srcT, dstT, cntT, Yout,
                srcv, dstv, cntv, rows_a, rows_b, dinvv, agg, sem, sem2):
    c = lax.axis_index("c")
    s = lax.axis_index("s")
    t = c * NS + s
    zeros_f = jnp.zeros((L,), jnp.float32)

    def zrow(j, _):
        for k in range(NHID // L):
            agg[j, pl.ds(k * L, L)] = zeros_f
        return 0
    lax.fori_loop(0, AGG_ROWS, zrow, 0)

    pltpu.sync_copy(cntT.at[pl.ds(t * CSLOT, CSLOT)], cntv)
    total = jnp.max(cntv[pl.ds(0, L)])
    nseg = jnp.max(cntv[pl.ds(L, L)])

    def accumulate(j, buf):
        def acc_group(e2, _):
            dvec = dstv[pl.ds(j * BATCH + e2 * L, L)]
            dvals = [dvec[lane] for lane in range(L)]
            prev = None
            for lane in range(L):
                for kk in range(2):
                    cur = []
                    for k8 in range(8):
                        if prev is not None:
                            pv, pd, pk = prev
                            plsc.addupdate(
                                agg.at[pd, pl.ds(pk * 8 * L + k8 * L, L)],
                                pv[k8])
                        cur.append(
                            buf[e2 * L + lane, pl.ds(kk * 8 * L + k8 * L, L)])
                    prev = (cur, dvals[lane], kk)
            pv, pd, pk = prev
            for k8 in range(8):
                plsc.addupdate(agg.at[pd, pl.ds(pk * 8 * L + k8 * L, L)],
                               pv[k8])
            return 0
        lax.fori_loop(0, BATCH // L, acc_group, 0)

    def seg_loop(sg, _):
        off = pl.multiple_of(t * REG + sg * SEG, CSLOT)
        pltpu.sync_copy(srcT.at[pl.ds(off, SEG)], srcv)
        pltpu.sync_copy(dstT.at[pl.ds(off, SEG)], dstv)
        nb = jnp.minimum(NBSEG, (total - sg * SEG) >> 5)

        @pl.when(nb > 0)
        def _():
            pltpu.async_copy(Yin.at[srcv.at[pl.ds(0, BATCH)]], rows_a, sem)

        def pair_loop(p, _):
            for b, (buf, sb) in enumerate(((rows_a, sem), (rows_b, sem2))):
                j = p * 2 + b

                @pl.when(j < nb)
                def _():
                    pltpu.make_async_copy(Yin.at[srcv.at[pl.ds(0, BATCH)]],
                                          buf, sb).wait()
                    nxt = jnp.minimum(j + 1, nb - 1) * BATCH
                    nbuf = rows_b if b == 0 else rows_a
                    nsem = sem2 if b == 0 else sem

                    @pl.when(j + 1 < nb)
                    def _():
                        pltpu.async_copy(Yin.at[srcv.at[pl.ds(nxt, BATCH)]],
                                         nbuf, nsem)
                    accumulate(j, buf)
            return 0
        lax.fori_loop(0, (NBSEG + 1) // 2, pair_loop, 0)
        return 0
    lax.fori_loop(0, nseg, seg_loop, 0)

    g0 = t * ROWS_T
    pltpu.sync_copy(dinvT.at[pl.ds(t * DSLOT, DSLOT)], dinvv)
    def ublock(u, _):
        g = g0 + u * BATCH
        pltpu.sync_copy(Yin.at[pl.ds(g, BATCH)], rows_a)
        pltpu.sync_copy(h0.at[pl.ds(g, BATCH)], rows_b)

        def upd(q, _):
            dinv = dinvv[pl.ds(u * BATCH + q * L, L)]
            for rr in range(L):
                d = ALPHA * dinv[rr]
                r = q * L + rr

                def upd_half(kk, _):
                    css = [pl.ds(kk * 8 * L + k8 * L, L) for k8 in range(8)]
                    ys = [rows_a[r, cs] for cs in css]
                    hs = [rows_b[r, cs] for cs in css]
                    ags = [agg[u * BATCH + r, cs] for cs in css]
                    for k8 in range(8):
                        nv = ((1.0 - ALPHA) * ys[k8]
                              + d * (LAMDA * ags[k8] + hs[k8]))
                        rows_b[r, css[k8]] = jnp.maximum(nv, 0.0)
                    return 0
                lax.fori_loop(0, 2, upd_half, 0)
            return 0
        lax.fori_loop(0, BATCH // L, upd, 0)
        pltpu.sync_copy(rows_b, Yout.at[pl.ds(g, BATCH)])
        return 0
    lax.fori_loop(0, ROWS_T // BATCH, ublock, 0)


_layer = pl.kernel(
    _layer_body,
    compiler_params=pltpu.CompilerParams(needs_layout_passes=False),
    out_type=jax.ShapeDtypeStruct((NPAD, NHID), jnp.float32),
    mesh=_mesh,
    scratch_types=[
        pltpu.VMEM((SEG,), jnp.int32),
        pltpu.VMEM((SEG,), jnp.int32),
        pltpu.VMEM((CSLOT,), jnp.int32),
        pltpu.VMEM((BATCH, NHID), jnp.float32),
        pltpu.VMEM((BATCH, NHID), jnp.float32),
        pltpu.VMEM((DSLOT,), jnp.float32),
        pltpu.VMEM((AGG_ROWS, NHID), jnp.float32),
        pltpu.SemaphoreType.DMA,
        pltpu.SemaphoreType.DMA,
    ],
)


def kernel(fea, adj, W_in, W_out):
    h0 = _matmul(fea, W_in, relu=True, out_rows=NPAD)
    adjp = jnp.pad(adj.reshape(2, NT, CHUNK),
                   ((0, 0), (0, 0), (0, CAP - CHUNK))).reshape(-1)
    srcT, dstT, cntT, dinvT = _prep(adjp)
    Y = h0
    for _ in range(NLAYERS):
        Y = _layer(Y, h0, dinvT, srcT, dstT, cntT)
    return _matmul(Y, W_out, relu=False, out_rows=N)
